# baseline (speedup 1.0000x reference)
import functools

import jax
import jax.numpy as jnp
from jax.experimental import pallas as pl

BSZ, L, D = 4, 512, 128
HEADS, OUT_CH = 2, 64


def _gat_kernel(x_ref, graph_ref, w_ref, att_src_ref, att_dst_ref, bias_ref,
                out_ref):
    x = x_ref[0]
    w = w_ref[...]
    h = jax.lax.dot_general(x, w, (((1,), (1,)), ((), ())),
                            preferred_element_type=jnp.float32)

    ii = jax.lax.broadcasted_iota(jnp.int32, (L, L), 0)
    jj = jax.lax.broadcasted_iota(jnp.int32, (L, L), 1)
    mask = (graph_ref[...] != 0.0) | (ii == jj)

    bias = bias_ref[...]
    outs = []
    for hd in range(HEADS):
        hh = h[:, hd * OUT_CH:(hd + 1) * OUT_CH]
        asrc = jnp.sum(hh * att_src_ref[hd:hd + 1, :], axis=1, keepdims=True)
        adst = jnp.sum(hh * att_dst_ref[hd:hd + 1, :], axis=1, keepdims=True)
        logits = asrc + adst.T
        logits = jnp.where(logits >= 0.0, logits, 0.2 * logits)
        neg = jnp.float32(-jnp.inf)
        m = jnp.max(jnp.where(mask, logits, neg), axis=0, keepdims=True)
        e = jnp.where(mask, jnp.exp(logits - m), 0.0)
        denom = jnp.sum(e, axis=0, keepdims=True)
        num = jax.lax.dot_general(e, hh, (((0,), (0,)), ((), ())),
                                  preferred_element_type=jnp.float32)
        outs.append(num / (denom.T + 1e-16))
    out = jnp.concatenate(outs, axis=1) + bias
    out_ref[0] = jnp.maximum(out, 0.0) + x


@jax.jit
def _gat(x, graph, W, att_src, att_dst, bias):
    bias2 = bias.reshape(1, HEADS * OUT_CH)
    return pl.pallas_call(
        _gat_kernel,
        grid=(BSZ,),
        in_specs=[
            pl.BlockSpec((1, L, D), lambda b: (b, 0, 0)),
            pl.BlockSpec((L, L), lambda b: (0, 0)),
            pl.BlockSpec((HEADS * OUT_CH, D), lambda b: (0, 0)),
            pl.BlockSpec((HEADS, OUT_CH), lambda b: (0, 0)),
            pl.BlockSpec((HEADS, OUT_CH), lambda b: (0, 0)),
            pl.BlockSpec((1, HEADS * OUT_CH), lambda b: (0, 0)),
        ],
        out_specs=pl.BlockSpec((1, L, D), lambda b: (b, 0, 0)),
        out_shape=jax.ShapeDtypeStruct((BSZ, L, HEADS * OUT_CH), jnp.float32),
    )(x, graph, W, att_src, att_dst, bias2)


def kernel(x, graph, W, att_src, att_dst, bias):
    return _gat(x, graph, W, att_src, att_dst, bias)


# speedup vs baseline: 1.2683x; 1.2683x over previous
import jax
import jax.numpy as jnp
from jax.experimental import pallas as pl

BSZ, L, D = 4, 512, 128
HEADS, OUT_CH = 2, 64


def _gat_kernel(x_ref, graph_ref, w_ref, att_src_ref, att_dst_ref, bias_ref,
                out_ref):
    x = x_ref[0]
    w = w_ref[...]
    h = jax.lax.dot_general(x, w, (((1,), (1,)), ((), ())),
                            preferred_element_type=jnp.float32)

    ii = jax.lax.broadcasted_iota(jnp.int32, (L, L), 0)
    jj = jax.lax.broadcasted_iota(jnp.int32, (L, L), 1)
    maskf = ((graph_ref[...] != 0.0) | (ii == jj)).astype(jnp.float32)

    ones_col = jnp.ones((L, 1), dtype=jnp.float32)
    bias = bias_ref[...]
    outs = []
    for hd in range(HEADS):
        hh = h[:, hd * OUT_CH:(hd + 1) * OUT_CH]
        s = jnp.sum(hh * att_src_ref[hd:hd + 1, :], axis=1, keepdims=True)
        d = jnp.sum(hh * att_dst_ref[hd:hd + 1, :], axis=1, keepdims=True)
        dT = d.T
        S = jnp.max(s)
        u = S + dT
        K = jnp.where(u >= 0.0, u, 0.2 * u)
        A = jnp.exp(s - S)
        C = jnp.exp(0.2 * (s - S))
        B = jnp.exp(dT + S - K)
        Dg = jnp.exp(0.2 * (dT + S) - K)
        v = s + dT
        e = jnp.where(v >= 0.0, A * B, C * Dg) * maskf
        hh1 = jnp.concatenate([hh, ones_col], axis=1)
        num = jax.lax.dot_general(e, hh1, (((0,), (0,)), ((), ())),
                                  preferred_element_type=jnp.float32)
        outs.append(num[:, :OUT_CH] / (num[:, OUT_CH:] + 1e-16))
    out = jnp.concatenate(outs, axis=1) + bias
    out_ref[0] = jnp.maximum(out, 0.0) + x


@jax.jit
def _gat(x, graph, W, att_src, att_dst, bias):
    bias2 = bias.reshape(1, HEADS * OUT_CH)
    return pl.pallas_call(
        _gat_kernel,
        grid=(BSZ,),
        in_specs=[
            pl.BlockSpec((1, L, D), lambda b: (b, 0, 0)),
            pl.BlockSpec((L, L), lambda b: (0, 0)),
            pl.BlockSpec((HEADS * OUT_CH, D), lambda b: (0, 0)),
            pl.BlockSpec((HEADS, OUT_CH), lambda b: (0, 0)),
            pl.BlockSpec((HEADS, OUT_CH), lambda b: (0, 0)),
            pl.BlockSpec((1, HEADS * OUT_CH), lambda b: (0, 0)),
        ],
        out_specs=pl.BlockSpec((1, L, D), lambda b: (b, 0, 0)),
        out_shape=jax.ShapeDtypeStruct((BSZ, L, HEADS * OUT_CH), jnp.float32),
    )(x, graph, W, att_src, att_dst, bias2)


def kernel(x, graph, W, att_src, att_dst, bias):
    return _gat(x, graph, W, att_src, att_dst, bias)
